# Initial kernel scaffold; baseline (speedup 1.0000x reference)
import jax
import jax.numpy as jnp
from jax.experimental import pallas as pl


def kernel(input, adj, weight_self, weight_neighbor, bias):
    raise NotImplementedError("write your pallas kernel here")


# speedup vs baseline: 1.0890x; 1.0890x over previous
import functools

import jax
import jax.numpy as jnp
from jax.experimental import pallas as pl
from jax.experimental.pallas import tpu as pltpu


def _gcn_block(x_ref, ws_ref, wn_ref, b_ref, adj_ref, out_ref, support_ref, *, bm):
    m = pl.program_id(0)

    @pl.when(m == 0)
    def _():
        support_ref[...] = jnp.dot(
            x_ref[...], wn_ref[...], preferred_element_type=jnp.float32
        )

    x_blk = x_ref[pl.ds(m * bm, bm), :]
    acc = jnp.dot(x_blk, ws_ref[...], preferred_element_type=jnp.float32)
    acc += jnp.dot(adj_ref[...], support_ref[...], preferred_element_type=jnp.float32)
    out_ref[...] = acc + b_ref[...]


def kernel(input, adj, weight_self, weight_neighbor, bias):
    n, d_in = input.shape
    d_out = weight_self.shape[1]
    bm = 400
    grid = (n // bm,)
    return pl.pallas_call(
        functools.partial(_gcn_block, bm=bm),
        grid=grid,
        in_specs=[
            pl.BlockSpec((n, d_in), lambda m: (0, 0)),
            pl.BlockSpec((d_in, d_out), lambda m: (0, 0)),
            pl.BlockSpec((d_in, d_out), lambda m: (0, 0)),
            pl.BlockSpec((1, d_out), lambda m: (0, 0)),
            pl.BlockSpec((bm, n), lambda m: (m, 0)),
        ],
        out_specs=pl.BlockSpec((bm, d_out), lambda m: (m, 0)),
        out_shape=jax.ShapeDtypeStruct((n, d_out), jnp.float32),
        scratch_shapes=[pltpu.VMEM((n, d_out), jnp.float32)],
    )(input, weight_self, weight_neighbor, bias.reshape(1, -1), adj)
